# baseline (speedup 1.0000x reference)
import functools

import jax
import jax.numpy as jnp
from jax import lax
from jax.experimental import pallas as pl
from jax.experimental.pallas import tpu as pltpu
from jax.experimental.pallas import tpu_sc as plsc

NUM_USERS = 1000000
NUM_ITEMS = 1000000
EMB_DIM = 64
BATCH = 16384

_NC = 2
_NS = 16
_L = 16
_NW = _NC * _NS
_BPW = BATCH // _NW
_GROUPS = _BPW // _L


def _rsqrt16(s):
    s = jnp.maximum(s, jnp.float32(1e-24))
    i = lax.bitcast_convert_type(s, jnp.int32)
    i = jnp.int32(0x5F3759DF) - lax.shift_right_logical(i, 1)
    y = lax.bitcast_convert_type(i, jnp.float32)
    for _ in range(2):
        y = y * (jnp.float32(1.5) - jnp.float32(0.5) * s * y * y)
    return y


def _shuffle_xor(x, lanes, k):
    idx = lax.bitwise_xor(lanes, jnp.int32(k))
    return lax.gather(
        x, idx[:, None],
        dimension_numbers=lax.GatherDimensionNumbers(
            offset_dims=(), collapsed_slice_dims=(0,), start_index_map=(0,)),
        slice_sizes=(1,),
        mode=lax.GatherScatterMode.PROMISE_IN_BOUNDS)


def _normalize_rows(rows_v):
    lanes = lax.iota(jnp.int32, _L)
    _QS = EMB_DIM // _L
    _UNROLL = 4

    def row_body(rr, _):
        for u in range(_UNROLL):
            r = rr * _UNROLL + u
            vs = [rows_v[r, pl.ds(q * _L, _L)] for q in range(_QS)]
            acc = vs[0] * vs[0]
            for q in range(1, _QS):
                acc = acc + vs[q] * vs[q]
            for k in (1, 2, 4, 8):
                acc = acc + _shuffle_xor(acc, lanes, k)
            inv = _rsqrt16(acc)
            for q in range(_QS):
                rows_v[r, pl.ds(q * _L, _L)] = vs[q] * inv
        return _

    lax.fori_loop(0, _BPW // _UNROLL, row_body, None)


@functools.cache
def _make_sc_kernel():
    mesh = plsc.VectorSubcoreMesh(core_axis_name="c", subcore_axis_name="s")

    @functools.partial(
        pl.kernel,
        mesh=mesh,
        compiler_params=pltpu.CompilerParams(use_tc_tiling_on_sc=False),
        out_type=jax.ShapeDtypeStruct((2, BATCH, EMB_DIM), jnp.float32),
        scratch_types=[
            pltpu.VMEM((_BPW,), jnp.int32),
            pltpu.VMEM((_BPW,), jnp.int32),
            pltpu.VMEM((_BPW, EMB_DIM), jnp.float32),
            pltpu.VMEM((_BPW, EMB_DIM), jnp.float32),
            pltpu.SemaphoreType.DMA,
            pltpu.SemaphoreType.DMA,
        ],
    )
    def two_tower(user_idx, item_idx, user_table, item_table, out,
                  uidx_v, iidx_v, urows_v, irows_v, usem, isem):
        wid = lax.axis_index("s") * _NC + lax.axis_index("c")
        base = wid * _BPW

        pltpu.sync_copy(user_idx.at[pl.ds(base, _BPW)], uidx_v)
        ucopy = pltpu.async_copy(user_table.at[uidx_v], urows_v, usem)
        pltpu.sync_copy(item_idx.at[pl.ds(base, _BPW)], iidx_v)
        icopy = pltpu.async_copy(item_table.at[iidx_v], irows_v, isem)

        ucopy.wait()
        _normalize_rows(urows_v)
        uout = pltpu.async_copy(urows_v, out.at[0, pl.ds(base, _BPW)], usem)

        icopy.wait()
        _normalize_rows(irows_v)
        pltpu.sync_copy(irows_v, out.at[1, pl.ds(base, _BPW)])
        uout.wait()

    return two_tower


def kernel(user_idx, item_idx, user_table, item_table):
    return _make_sc_kernel()(user_idx, item_idx, user_table, item_table)


# speedup vs baseline: 1.5604x; 1.5604x over previous
import functools

import jax
import jax.numpy as jnp
from jax import lax
from jax.experimental import pallas as pl
from jax.experimental.pallas import tpu as pltpu
from jax.experimental.pallas import tpu_sc as plsc

NUM_USERS = 1000000
NUM_ITEMS = 1000000
EMB_DIM = 64
BATCH = 16384

_NC = 2
_NS = 16
_L = 16
_NW = _NC * _NS
_BPW = BATCH // _NW
_GROUPS = _BPW // _L


def _rsqrt16(s):
    s = jnp.maximum(s, jnp.float32(1e-24))
    i = lax.bitcast_convert_type(s, jnp.int32)
    i = jnp.int32(0x5F3759DF) - lax.shift_right_logical(i, 1)
    y = lax.bitcast_convert_type(i, jnp.float32)
    for _ in range(2):
        y = y * (jnp.float32(1.5) - jnp.float32(0.5) * s * y * y)
    return y


def _shuffle_xor(x, lanes, k):
    idx = lax.bitwise_xor(lanes, jnp.int32(k))
    return lax.gather(
        x, idx[:, None],
        dimension_numbers=lax.GatherDimensionNumbers(
            offset_dims=(), collapsed_slice_dims=(0,), start_index_map=(0,)),
        slice_sizes=(1,),
        mode=lax.GatherScatterMode.PROMISE_IN_BOUNDS)


def _normalize_rows(rows_v):
    lanes = lax.iota(jnp.int32, _L)
    _QS = EMB_DIM // _L
    _UNROLL = 4

    def row_body(rr, _):
        for u in range(_UNROLL):
            r = rr * _UNROLL + u
            vs = [rows_v[r, pl.ds(q * _L, _L)] for q in range(_QS)]
            acc = vs[0] * vs[0]
            for q in range(1, _QS):
                acc = acc + vs[q] * vs[q]
            for k in (1, 2, 4, 8):
                acc = acc + _shuffle_xor(acc, lanes, k)
            inv = _rsqrt16(acc)
            for q in range(_QS):
                rows_v[r, pl.ds(q * _L, _L)] = vs[q] * inv
        return _

    lax.fori_loop(0, _BPW // _UNROLL, row_body, None)


@functools.cache
def _make_sc_kernel():
    mesh = plsc.VectorSubcoreMesh(core_axis_name="c", subcore_axis_name="s")

    @functools.partial(
        pl.kernel,
        mesh=mesh,
        out_type=jax.ShapeDtypeStruct((2, BATCH, EMB_DIM), jnp.float32),
        scratch_types=[
            pltpu.VMEM((_BPW,), jnp.int32),
            pltpu.VMEM((_BPW,), jnp.int32),
            pltpu.VMEM((_BPW, EMB_DIM), jnp.float32),
            pltpu.SemaphoreType.DMA,
            pltpu.SemaphoreType.DMA,
        ],
    )
    def two_tower(user_idx, item_idx, user_table, item_table, out,
                  uidx_v, iidx_v, rows_v, gsem, osem):
        wid = lax.axis_index("s") * _NC + lax.axis_index("c")
        base = wid * _BPW

        pltpu.sync_copy(user_idx.at[pl.ds(base, _BPW)], uidx_v)
        pltpu.sync_copy(item_idx.at[pl.ds(base, _BPW)], iidx_v)

        for tower, tab, idx_v in ((0, user_table, uidx_v),
                                  (1, item_table, iidx_v)):
            def issue(g, _, tab=tab, idx_v=idx_v):
                iv = idx_v[pl.ds(g * _L, _L)]
                for k in range(_L):
                    pltpu.async_copy(
                        tab.at[iv[k]], rows_v.at[g * _L + k], gsem)
                return _

            lax.fori_loop(0, _BPW // _L, issue, None)
            pltpu.make_async_copy(
                tab.at[pl.ds(0, _BPW)], rows_v, gsem).wait()
            _normalize_rows(rows_v)
            if tower == 1:
                pltpu.sync_copy(rows_v, out.at[tower, pl.ds(base, _BPW)])
            else:
                pltpu.sync_copy(rows_v, out.at[tower, pl.ds(base, _BPW)])

    return two_tower


def kernel(user_idx, item_idx, user_table, item_table):
    return _make_sc_kernel()(user_idx, item_idx, user_table, item_table)
